# baseline (speedup 1.0000x reference)
import functools

import jax
import jax.numpy as jnp
from jax import lax
from jax.experimental import pallas as pl
from jax.experimental.pallas import tpu as pltpu
from jax.experimental.pallas import tpu_sc as plsc

D = 64
EPS = 1e-5
NC = 2
NS = 16
NW = NC * NS
K = 128


def _lnorm_gather(total_n):
    n_per_w = total_n // NW
    n_blocks = n_per_w // K
    mesh = plsc.VectorSubcoreMesh(core_axis_name="c", subcore_axis_name="s")

    @functools.partial(
        pl.kernel,
        mesh=mesh,
        compiler_params=pltpu.CompilerParams(use_tc_tiling_on_sc=False),
        out_type=jax.ShapeDtypeStruct((total_n, D), jnp.float32),
        scratch_types=[
            pltpu.VMEM((K,), jnp.int32),
            pltpu.VMEM((K, D), jnp.float32),
            pltpu.VMEM((2, D), jnp.float32),
            pltpu.SemaphoreType.DMA,
        ],
    )
    def k(x_hbm, table_hbm, gamma_hbm, beta_hbm, out_hbm, idx_v, rows_v, gb_v, sem):
        wid = lax.axis_index("s") * NC + lax.axis_index("c")
        base0 = wid * n_per_w

        pltpu.sync_copy(gamma_hbm, gb_v.at[0])
        pltpu.sync_copy(beta_hbm, gb_v.at[1])
        gammas = [gb_v[0, pl.ds(16 * q, 16)] for q in range(4)]
        betas = [gb_v[1, pl.ds(16 * q, 16)] for q in range(4)]

        def block(b, carry):
            base = base0 + b * K
            pltpu.sync_copy(x_hbm.at[pl.ds(base, K)], idx_v)
            pltpu.async_copy(table_hbm.at[idx_v], rows_v, sem).wait()

            lane = lax.iota(jnp.int32, 16)
            perms = [(lane ^ k).reshape(16, 1) for k in (1, 2, 4, 8)]
            dnums = lax.GatherDimensionNumbers(
                offset_dims=(), collapsed_slice_dims=(0,), start_index_map=(0,)
            )

            def lanesum(v):
                for p in perms:
                    v = v + lax.gather(
                        v, p, dnums, (1,),
                        mode=lax.GatherScatterMode.PROMISE_IN_BOUNDS,
                    )
                return v

            def row(r, c):
                vs = [rows_v[r, pl.ds(16 * q, 16)] for q in range(4)]
                s = (vs[0] + vs[1]) + (vs[2] + vs[3])
                sq = (vs[0] * vs[0] + vs[1] * vs[1]) + (vs[2] * vs[2] + vs[3] * vs[3])
                tot = lanesum(s)
                tot2 = lanesum(sq)
                mean = tot * (1.0 / D)
                var = tot2 * (1.0 / D) - mean * mean
                xv = var + EPS
                i = lax.bitcast_convert_type(xv, jnp.int32)
                i = jnp.int32(0x5F3759DF) - lax.shift_right_logical(i, 1)
                y = lax.bitcast_convert_type(i, jnp.float32)
                half_x = 0.5 * xv
                for _ in range(3):
                    y = y * (1.5 - half_x * y * y)
                for q in range(4):
                    rows_v[r, pl.ds(16 * q, 16)] = (
                        (vs[q] - mean) * y * gammas[q] + betas[q]
                    )
                return c

            lax.fori_loop(0, K, row, 0, unroll=2)
            pltpu.sync_copy(rows_v, out_hbm.at[pl.ds(base, K)])
            return carry

        lax.fori_loop(0, n_blocks, block, 0)

    return k


def kernel(x, table, gamma, beta):
    b, s = x.shape
    total_n = b * s
    out = _lnorm_gather(total_n)(x.reshape(total_n), table, gamma, beta)
    return out.reshape(b, s, D)


# speedup vs baseline: 1.2222x; 1.2222x over previous
import functools

import jax
import jax.numpy as jnp
from jax import lax
from jax.experimental import pallas as pl
from jax.experimental.pallas import tpu as pltpu
from jax.experimental.pallas import tpu_sc as plsc

D = 64
EPS = 1e-5
NC = 2
NS = 16
NW = NC * NS
K = 128
NBUF = 4
PF = 2


def _lnorm_gather(total_n):
    n_per_w = total_n // NW
    n_blocks = n_per_w // K
    n_t = n_blocks // NBUF
    mesh = plsc.VectorSubcoreMesh(core_axis_name="c", subcore_axis_name="s")

    @functools.partial(
        pl.kernel,
        mesh=mesh,
        compiler_params=pltpu.CompilerParams(use_tc_tiling_on_sc=False),
        out_type=jax.ShapeDtypeStruct((total_n, D), jnp.float32),
        scratch_types=[
            pltpu.VMEM((n_per_w,), jnp.int32),
            pltpu.VMEM((NBUF, K, D), jnp.float32),
            pltpu.VMEM((2, D), jnp.float32),
            [pltpu.SemaphoreType.DMA] * NBUF,
            [pltpu.SemaphoreType.DMA] * NBUF,
        ],
    )
    def k(x_hbm, table_hbm, gamma_hbm, beta_hbm, out_hbm, idx_v, rows_v, gb_v,
          gsems, osems):
        wid = lax.axis_index("s") * NC + lax.axis_index("c")
        base0 = wid * n_per_w

        pltpu.sync_copy(gamma_hbm, gb_v.at[0])
        pltpu.sync_copy(beta_hbm, gb_v.at[1])
        pltpu.sync_copy(x_hbm.at[pl.ds(base0, n_per_w)], idx_v)
        gammas = [gb_v[0, pl.ds(16 * q, 16)] for q in range(4)]
        betas = [gb_v[1, pl.ds(16 * q, 16)] for q in range(4)]

        lane = lax.iota(jnp.int32, 16)
        perms = [(lane ^ kk).reshape(16, 1) for kk in (1, 2, 4, 8)]
        dnums = lax.GatherDimensionNumbers(
            offset_dims=(), collapsed_slice_dims=(0,), start_index_map=(0,)
        )

        def lanesum(v):
            for p in perms:
                v = v + lax.gather(
                    v, p, dnums, (1,),
                    mode=lax.GatherScatterMode.PROMISE_IN_BOUNDS,
                )
            return v

        def start_gather(blk, q):
            pltpu.async_copy(
                table_hbm.at[idx_v.at[pl.ds(blk * K, K)]],
                rows_v.at[q],
                gsems[q],
            )

        def wait_gather(q):
            pltpu.make_async_copy(
                table_hbm.at[idx_v.at[pl.ds(0, K)]], rows_v.at[q], gsems[q]
            ).wait()

        def start_write(blk, q):
            pltpu.async_copy(
                rows_v.at[q], out_hbm.at[pl.ds(base0 + blk * K, K)], osems[q]
            )

        def wait_write(q):
            pltpu.make_async_copy(
                rows_v.at[q], out_hbm.at[pl.ds(0, K)], osems[q]
            ).wait()

        def compute_block(p):
            def row(r, c):
                vs = [rows_v[p, r, pl.ds(16 * q, 16)] for q in range(4)]
                s = (vs[0] + vs[1]) + (vs[2] + vs[3])
                sq = (vs[0] * vs[0] + vs[1] * vs[1]) + (vs[2] * vs[2] + vs[3] * vs[3])
                tot = lanesum(s)
                tot2 = lanesum(sq)
                mean = tot * (1.0 / D)
                var = tot2 * (1.0 / D) - mean * mean
                xv = var + EPS
                i = lax.bitcast_convert_type(xv, jnp.int32)
                i = jnp.int32(0x5F3759DF) - lax.shift_right_logical(i, 1)
                y = lax.bitcast_convert_type(i, jnp.float32)
                half_x = 0.5 * xv
                for _ in range(3):
                    y = y * (1.5 - half_x * y * y)
                for q in range(4):
                    rows_v[p, r, pl.ds(16 * q, 16)] = (
                        (vs[q] - mean) * y * gammas[q] + betas[q]
                    )
                return c

            lax.fori_loop(0, K, row, 0, unroll=2)

        start_gather(0, 0)
        start_gather(1, 1)

        def body(t, carry):
            for p in range(NBUF):
                b = t * NBUF + p
                q = (p + PF) % NBUF
                if p < PF:
                    @pl.when(t > 0)
                    def _():
                        wait_write(q)
                else:
                    wait_write(q)
                start_gather(b + PF, q)
                wait_gather(p)
                compute_block(p)
                start_write(b, p)
            return carry

        lax.fori_loop(0, n_t - 1, body, 0)

        for p in range(NBUF):
            b = (n_t - 1) * NBUF + p
            q = (p + PF) % NBUF
            if p < PF:
                wait_write(q)
                start_gather(b + PF, q)
            wait_gather(p)
            compute_block(p)
            start_write(b, p)

        for q in range(NBUF):
            wait_write(q)

    return k


def kernel(x, table, gamma, beta):
    b, s = x.shape
    total_n = b * s
    out = _lnorm_gather(total_n)(x.reshape(total_n), table, gamma, beta)
    return out.reshape(b, s, D)


# speedup vs baseline: 1.3787x; 1.1280x over previous
import functools

import jax
import jax.numpy as jnp
from jax import lax
from jax.experimental import pallas as pl
from jax.experimental.pallas import tpu as pltpu
from jax.experimental.pallas import tpu_sc as plsc

D = 64
EPS = 1e-5
NC = 2
NS = 16
NW = NC * NS
K = 128
NBUF = 4
PF = 2


def _lnorm_gather(total_n):
    n_per_w = total_n // NW
    n_blocks = n_per_w // K
    n_t = n_blocks // NBUF
    mesh = plsc.VectorSubcoreMesh(core_axis_name="c", subcore_axis_name="s")

    @functools.partial(
        pl.kernel,
        mesh=mesh,
        compiler_params=pltpu.CompilerParams(use_tc_tiling_on_sc=False),
        out_type=jax.ShapeDtypeStruct((total_n, D), jnp.float32),
        scratch_types=[
            pltpu.VMEM((n_per_w,), jnp.int32),
            pltpu.VMEM((NBUF, K, D), jnp.float32),
            pltpu.VMEM((2, D), jnp.float32),
            [pltpu.SemaphoreType.DMA] * NBUF,
            [pltpu.SemaphoreType.DMA] * NBUF,
        ],
    )
    def k(x_hbm, table_hbm, gamma_hbm, beta_hbm, out_hbm, idx_v, rows_v, gb_v,
          gsems, osems):
        wid = lax.axis_index("s") * NC + lax.axis_index("c")
        base0 = wid * n_per_w

        pltpu.sync_copy(gamma_hbm, gb_v.at[0])
        pltpu.sync_copy(beta_hbm, gb_v.at[1])
        pltpu.sync_copy(x_hbm.at[pl.ds(base0, n_per_w)], idx_v)
        gammas = [gb_v[0, pl.ds(16 * q, 16)] for q in range(4)]
        betas = [gb_v[1, pl.ds(16 * q, 16)] for q in range(4)]

        lane = lax.iota(jnp.int32, 16)
        perms = [(lane ^ kk).reshape(16, 1) for kk in (1, 2, 4, 8)]
        dnums = lax.GatherDimensionNumbers(
            offset_dims=(), collapsed_slice_dims=(0,), start_index_map=(0,)
        )

        def lanesum(v):
            for p in perms:
                v = v + lax.gather(
                    v, p, dnums, (1,),
                    mode=lax.GatherScatterMode.PROMISE_IN_BOUNDS,
                )
            return v

        def start_gather(blk, q):
            pltpu.async_copy(
                table_hbm.at[idx_v.at[pl.ds(blk * K, K)]],
                rows_v.at[q],
                gsems[q],
            )

        def wait_gather(q):
            pltpu.make_async_copy(
                table_hbm.at[idx_v.at[pl.ds(0, K)]], rows_v.at[q], gsems[q]
            ).wait()

        def start_write(blk, q):
            pltpu.async_copy(
                rows_v.at[q], out_hbm.at[pl.ds(base0 + blk * K, K)], osems[q]
            )

        def wait_write(q):
            pltpu.make_async_copy(
                rows_v.at[q], out_hbm.at[pl.ds(0, K)], osems[q]
            ).wait()

        def compute_block(p):
            def row(r, c):
                vs = [rows_v[p, r, pl.ds(16 * q, 16)] for q in range(4)]
                s = (vs[0] + vs[1]) + (vs[2] + vs[3])
                sq = (vs[0] * vs[0] + vs[1] * vs[1]) + (vs[2] * vs[2] + vs[3] * vs[3])
                tot = lanesum(s)
                tot2 = lanesum(sq)
                mean = tot * (1.0 / D)
                var = tot2 * (1.0 / D) - mean * mean
                xv = var + EPS
                i = lax.bitcast_convert_type(xv, jnp.int32)
                i = jnp.int32(0x5F3759DF) - lax.shift_right_logical(i, 1)
                y = lax.bitcast_convert_type(i, jnp.float32)
                half_x = 0.5 * xv
                for _ in range(3):
                    y = y * (1.5 - half_x * y * y)
                for q in range(4):
                    rows_v[p, r, pl.ds(16 * q, 16)] = (
                        (vs[q] - mean) * y * gammas[q] + betas[q]
                    )
                return c

            lax.fori_loop(0, K, row, 0, unroll=4)

        start_gather(0, 0)
        start_gather(1, 1)

        def body(t, carry):
            for p in range(NBUF):
                b = t * NBUF + p
                q = (p + PF) % NBUF
                if p < PF:
                    @pl.when(t > 0)
                    def _():
                        wait_write(q)
                else:
                    wait_write(q)
                start_gather(b + PF, q)
                wait_gather(p)
                compute_block(p)
                start_write(b, p)
            return carry

        lax.fori_loop(0, n_t - 1, body, 0)

        for p in range(NBUF):
            b = (n_t - 1) * NBUF + p
            q = (p + PF) % NBUF
            if p < PF:
                wait_write(q)
                start_gather(b + PF, q)
            wait_gather(p)
            compute_block(p)
            start_write(b, p)

        for q in range(NBUF):
            wait_write(q)

    return k


def kernel(x, table, gamma, beta):
    b, s = x.shape
    total_n = b * s
    out = _lnorm_gather(total_n)(x.reshape(total_n), table, gamma, beta)
    return out.reshape(b, s, D)
